# baseline (speedup 1.0000x reference)
import functools
import jax
import jax.numpy as jnp
from jax import lax
from jax.experimental import pallas as pl
from jax.experimental.pallas import tpu as pltpu
from jax.experimental.pallas import tpu_sc as plsc

_HP = 16
_N = _HP * _HP
_NP = 49
_ROWS = _NP * _N
_K = 9
_D4 = 32
_NEG = -3.0e38

_NW = 32
_RPW = _ROWS // _NW
_NG = 25


def _sim_body(scal_ref, x_ref, fw_ref, fb_ref, comb_ref, xflat_ref, d_scr):
    p = pl.program_id(0)
    sigma = scal_ref[0, 0]
    alpha = scal_ref[0, 1]

    @pl.when(p == 0)
    def _():
        ni = lax.broadcasted_iota(jnp.int32, (_N, _N), 0)
        mi = lax.broadcasted_iota(jnp.int32, (_N, _N), 1)
        dr = (ni // _HP) - (mi // _HP)
        dc = (ni % _HP) - (mi % _HP)
        d2 = (dr * dr + dc * dc).astype(jnp.float32)
        d_scr[...] = jnp.exp(d2 * (-1.0 / (2.0 * sigma * sigma)))

    Dm = d_scr[...]
    xm = x_ref[0]
    ft = lax.dot_general(fw_ref[...], xm, (((1,), (0,)), ((), ())),
                         preferred_element_type=jnp.float32)
    ft = ft + fb_ref[...]
    nsq = jnp.sum(ft * ft, axis=0)[None, :]
    inv = 1.0 / jnp.maximum(jnp.sqrt(nsq), 1e-8)
    ftn = ft * inv
    S = lax.dot_general(ftn, ftn, (((0,), (0,)), ((), ())),
                        preferred_element_type=jnp.float32)
    comb_ref[...] = (alpha * S + (1.0 - alpha) * Dm)[None]
    xflat_ref[...] = ft.T[None]


def _proj_body(agg_ref, pw_ref, pb_ref, out_ref):
    y = lax.dot_general(pw_ref[...], agg_ref[0], (((1,), (1,)), ((), ())),
                        preferred_element_type=jnp.float32)
    out_ref[...] = (y + pb_ref[...])[None]


def _sc_mid_body(comb_hbm, xflat_hbm, params_hbm, out_hbm,
                 params_v, rowb0, rowb1, idx0a, idx0b, idx1a, idx1b,
                 rowsg0, rowsg1, outv, sem0, sem1, semg):
    wid = lax.axis_index("s") * 2 + lax.axis_index("c")
    base = wid * _RPW
    pltpu.sync_copy(params_hbm, params_v)

    def P(i):
        return params_v[i]

    iot = lax.iota(jnp.int32, 16)

    def topk2(buf):
        init = ([jnp.full((16,), _NEG, jnp.float32) for _ in range(2 * _K)]
                + [jnp.zeros((16,), jnp.int32) for _ in range(2 * _K)])

        def col_body(j, st):
            ts = list(st[:2 * _K])
            tis = list(st[2 * _K:])
            jv = jnp.full((16,), j, jnp.int32)
            va = plsc.load_gather(buf, [iot, jv])
            vb = plsc.load_gather(buf, [iot + 16, jv])
            vs = [va, vb]
            vis = [jv, jv]
            for g in range(2):
                v = vs[g]
                vi = vis[g]
                for s in range(_K):
                    k = g * _K + s
                    take = v > ts[k]
                    nt = jnp.where(take, v, ts[k])
                    nti = jnp.where(take, vi, tis[k])
                    v = jnp.where(take, ts[k], v)
                    vi = jnp.where(take, tis[k], vi)
                    ts[k] = nt
                    tis[k] = nti
            return tuple(ts) + tuple(tis)

        st = lax.fori_loop(0, _N, col_body, tuple(init))
        return st[:2 * _K], st[2 * _K:]

    def edge_weights(g0, ts, tis):
        g_vec = iot + jnp.full((16,), g0, jnp.int32)
        n_vec = lax.rem(g_vec, _N)
        pbase_vec = g_vec - n_vec
        rn = n_vec >> 4
        cn = n_vec & 15
        wes = []
        for s in range(_K):
            ri = tis[s] >> 4
            ci = tis[s] & 15
            dr = rn - ri
            dc = cn - ci
            d2 = (dr * dr + dc * dc).astype(jnp.float32)
            sd = jnp.exp(d2 * P(0))
            sf = (ts[s] - sd * P(1)) * P(2)
            tot = P(19)
            for i in range(4):
                h = sf * P(3 + 2 * i) + sd * P(4 + 2 * i) + P(11 + i)
                h = h / (1.0 + jnp.exp(-h))
                tot = tot + h * P(15 + i)
            wes.append(1.0 / (1.0 + jnp.exp(-tot)))
        wsum = wes[0]
        for s in range(1, _K):
            wsum = wsum + wes[s]
        winv = 1.0 / (wsum + 1e-12)
        return [we * winv for we in wes], pbase_vec

    def issue_gather(pbase_vec, tis, idxa, idxb, rowsg):
        for s in range(5):
            idxa[pl.ds(s * 16, 16)] = pbase_vec + tis[s]
        for s in range(5, _K):
            idxb[pl.ds((s - 5) * 16, 16)] = pbase_vec + tis[s]
        cp_a = pltpu.async_copy(xflat_hbm.at[idxa], rowsg.at[pl.ds(0, 80)], semg)
        cp_b = pltpu.async_copy(xflat_hbm.at[idxb], rowsg.at[pl.ds(80, 64)], semg)
        return cp_a, cp_b

    def aggregate(r0, wns, rowsg):
        rr = r0 + iot
        for d in range(_D4):
            dv = jnp.full((16,), d, jnp.int32)
            acc = wns[0] * plsc.load_gather(rowsg, [iot, dv])
            for s in range(1, _K):
                acc = acc + wns[s] * plsc.load_gather(rowsg, [iot + s * 16, dv])
            plsc.store_scatter(outv, [rr, dv], acc)

    def process_pair(buf, r0):
        ts2, tis2 = topk2(buf)
        wns_a, pb_a = edge_weights(base + r0, ts2[:_K], tis2[:_K])
        wns_b, pb_b = edge_weights(base + r0 + 16, ts2[_K:], tis2[_K:])
        cpa0, cpa1 = issue_gather(pb_a, tis2[:_K], idx0a, idx0b, rowsg0)
        cpb0, cpb1 = issue_gather(pb_b, tis2[_K:], idx1a, idx1b, rowsg1)
        cpa0.wait()
        cpa1.wait()
        aggregate(r0, wns_a, rowsg0)
        cpb0.wait()
        cpb1.wait()
        aggregate(r0 + 16, wns_b, rowsg1)

    cp_first = pltpu.async_copy(comb_hbm.at[pl.ds(base, 32)],
                                rowb0, sem0)
    cp_second = pltpu.async_copy(comb_hbm.at[pl.ds(base + 32, 32)],
                                 rowb1, sem1)
    cp_first.wait()

    def iter_body(i, carry):
        process_pair(rowb0, 64 * i)

        @pl.when(i < 5)
        def _():
            pltpu.async_copy(comb_hbm.at[pl.ds(base + 64 * i + 64, 32)],
                             rowb0, sem0)

        pltpu.make_async_copy(comb_hbm.at[pl.ds(base, 32)], rowb1, sem1).wait()
        process_pair(rowb1, 64 * i + 32)

        @pl.when(i < 5)
        def _():
            pltpu.async_copy(comb_hbm.at[pl.ds(base + 64 * i + 96, 32)],
                             rowb1, sem1)

        @pl.when(i < 5)
        def _():
            pltpu.make_async_copy(comb_hbm.at[pl.ds(base, 32)],
                                  rowb0, sem0).wait()
        return carry

    lax.fori_loop(0, 6, iter_body, 0)

    pltpu.sync_copy(comb_hbm.at[pl.ds(base + _RPW - 16, 16)],
                    rowb0.at[pl.ds(0, 16)])
    r0 = _RPW - 16
    g_last = base + r0
    init = ([jnp.full((16,), _NEG, jnp.float32) for _ in range(_K)]
            + [jnp.zeros((16,), jnp.int32) for _ in range(_K)])

    def col_body1(j, st):
        ts = list(st[:_K])
        tis = list(st[_K:])
        jv = jnp.full((16,), j, jnp.int32)
        v = plsc.load_gather(rowb0, [iot, jv])
        vi = jv
        for s in range(_K):
            take = v > ts[s]
            nt = jnp.where(take, v, ts[s])
            nti = jnp.where(take, vi, tis[s])
            v = jnp.where(take, ts[s], v)
            vi = jnp.where(take, tis[s], vi)
            ts[s] = nt
            tis[s] = nti
        return tuple(ts) + tuple(tis)

    st = lax.fori_loop(0, _N, col_body1, tuple(init))
    wns_l, pb_l = edge_weights(g_last, st[:_K], st[_K:])
    cpl0, cpl1 = issue_gather(pb_l, st[_K:], idx0a, idx0b, rowsg0)
    cpl0.wait()
    cpl1.wait()
    aggregate(r0, wns_l, rowsg0)

    pltpu.sync_copy(outv, out_hbm.at[pl.ds(base, _RPW)])


_sc_mid = functools.partial(
    pl.kernel,
    out_type=jax.ShapeDtypeStruct((_ROWS, _D4), jnp.float32),
    mesh=plsc.VectorSubcoreMesh(core_axis_name="c", subcore_axis_name="s"),
    compiler_params=pltpu.CompilerParams(use_tc_tiling_on_sc=False,
                                         needs_layout_passes=False),
    scratch_types=[
        pltpu.VMEM((24, 16), jnp.float32),
        pltpu.VMEM((32, _N), jnp.float32),
        pltpu.VMEM((32, _N), jnp.float32),
        pltpu.VMEM((80,), jnp.int32),
        pltpu.VMEM((64,), jnp.int32),
        pltpu.VMEM((80,), jnp.int32),
        pltpu.VMEM((64,), jnp.int32),
        pltpu.VMEM((144, _D4), jnp.float32),
        pltpu.VMEM((144, _D4), jnp.float32),
        pltpu.VMEM((_RPW, _D4), jnp.float32),
        pltpu.SemaphoreType.DMA,
        pltpu.SemaphoreType.DMA,
        pltpu.SemaphoreType.DMA,
    ],
)(_sc_mid_body)


@jax.jit
def kernel(x_in, sigma, alpha, f_w, f_b, p_w, p_b, mlp_w1, mlp_b1, mlp_w2, mlp_b2):
    B, C, H, Wd = x_in.shape
    ws = 7
    scal = jnp.stack([sigma, alpha]).reshape(1, 2).astype(jnp.float32)
    pvec = jnp.concatenate([
        jnp.stack([
            -1.0 / (2.0 * sigma * sigma),
            1.0 - alpha,
            1.0 / alpha,
        ]),
        mlp_w1.reshape(-1), mlp_b1.reshape(-1),
        mlp_w2.reshape(-1), mlp_b2.reshape(-1),
        jnp.zeros((4,), jnp.float32),
    ]).astype(jnp.float32)
    params = jnp.tile(pvec.reshape(-1, 1), (1, 16))

    xp = x_in.reshape(C, ws, _HP, ws, _HP).transpose(1, 3, 0, 2, 4).reshape(_NP, C, _N)

    comb, xflat = pl.pallas_call(
        _sim_body,
        grid=(_NP,),
        in_specs=[
            pl.BlockSpec((1, 2), lambda p: (0, 0), memory_space=pltpu.SMEM),
            pl.BlockSpec((1, C, _N), lambda p: (p, 0, 0)),
            pl.BlockSpec((_D4, C), lambda p: (0, 0)),
            pl.BlockSpec((_D4, 1), lambda p: (0, 0)),
        ],
        out_specs=[
            pl.BlockSpec((1, _N, _N), lambda p: (p, 0, 0)),
            pl.BlockSpec((1, _N, _D4), lambda p: (p, 0, 0)),
        ],
        out_shape=[
            jax.ShapeDtypeStruct((_NP, _N, _N), jnp.float32),
            jax.ShapeDtypeStruct((_NP, _N, _D4), jnp.float32),
        ],
        scratch_shapes=[pltpu.VMEM((_N, _N), jnp.float32)],
    )(scal, xp, f_w, f_b.reshape(_D4, 1))

    out32 = _sc_mid(comb.reshape(_ROWS, _N), xflat.reshape(_ROWS, _D4), params)

    out = pl.pallas_call(
        _proj_body,
        grid=(_NP,),
        in_specs=[
            pl.BlockSpec((1, _N, _D4), lambda p: (p, 0, 0)),
            pl.BlockSpec((C, _D4), lambda p: (0, 0)),
            pl.BlockSpec((C, 1), lambda p: (0, 0)),
        ],
        out_specs=pl.BlockSpec((1, C, _N), lambda p: (p, 0, 0)),
        out_shape=jax.ShapeDtypeStruct((_NP, C, _N), jnp.float32),
    )(out32.reshape(_NP, _N, _D4), p_w, p_b.reshape(C, 1))

    out = out.reshape(ws, ws, C, _HP, _HP).transpose(2, 0, 3, 1, 4).reshape(B, C, H * Wd)
    return out


# speedup vs baseline: 1.5280x; 1.5280x over previous
import functools
import jax
import jax.numpy as jnp
from jax import lax
from jax.experimental import pallas as pl
from jax.experimental.pallas import tpu as pltpu
from jax.experimental.pallas import tpu_sc as plsc

_HP = 16
_N = _HP * _HP
_NP = 49
_ROWS = _NP * _N
_K = 9
_D4 = 32
_NEG = -3.0e38

_NW = 32
_RPW = _ROWS // _NW
_NG = 25


def _sim_body(scal_ref, x_ref, fw_ref, fb_ref, comb_ref, xflat_ref, d_scr):
    p = pl.program_id(0)
    sigma = scal_ref[0, 0]
    alpha = scal_ref[0, 1]

    @pl.when(p == 0)
    def _():
        ni = lax.broadcasted_iota(jnp.int32, (_N, _N), 0)
        mi = lax.broadcasted_iota(jnp.int32, (_N, _N), 1)
        dr = (ni // _HP) - (mi // _HP)
        dc = (ni % _HP) - (mi % _HP)
        d2 = (dr * dr + dc * dc).astype(jnp.float32)
        d_scr[...] = jnp.exp(d2 * (-1.0 / (2.0 * sigma * sigma)))

    Dm = d_scr[...]
    xm = x_ref[0]
    ft = lax.dot_general(fw_ref[...], xm, (((1,), (0,)), ((), ())),
                         preferred_element_type=jnp.float32)
    ft = ft + fb_ref[...]
    nsq = jnp.sum(ft * ft, axis=0)[None, :]
    inv = 1.0 / jnp.maximum(jnp.sqrt(nsq), 1e-8)
    ftn = ft * inv
    S = lax.dot_general(ftn, ftn, (((0,), (0,)), ((), ())),
                        preferred_element_type=jnp.float32)
    comb_ref[...] = (alpha * S + (1.0 - alpha) * Dm)[None]
    xflat_ref[...] = ft.T[None]


def _proj_body(agg_ref, pw_ref, pb_ref, out_ref):
    y = lax.dot_general(pw_ref[...], agg_ref[0], (((1,), (1,)), ((), ())),
                        preferred_element_type=jnp.float32)
    out_ref[...] = (y + pb_ref[...])[None]


def _sc_mid_body(comb_hbm, xflat_hbm, params_hbm, out_hbm,
                 params_v, bta0, btb0, bta1, btb1,
                 idx0a, idx0b, idx1a, idx1b,
                 rowsg0, rowsg1, outvA, outvB, semA, semB, semg, semoA, semoB):
    wid = lax.axis_index("s") * 2 + lax.axis_index("c")
    pltpu.sync_copy(params_hbm, params_v)

    def P(i):
        return params_v[i]

    iot = lax.iota(jnp.int32, 16)

    def grp_src(grp):
        g0 = grp * 16
        pstart = (grp >> 4) * _N
        n0 = g0 - pstart
        return comb_hbm.at[pl.ds(pstart, _N), pl.ds(n0, 16)]

    def topk2(bufa, bufb):
        init = ([jnp.full((16,), _NEG, jnp.float32) for _ in range(2 * _K)]
                + [jnp.zeros((16,), jnp.int32) for _ in range(2 * _K)])

        def col_body(j, st):
            ts = list(st[:2 * _K])
            tis = list(st[2 * _K:])
            jv = jnp.full((16,), j, jnp.int32)
            vs = [bufa[j], bufb[j]]
            vis = [jv, jv]
            for g in range(2):
                v = vs[g]
                vi = vis[g]
                for s in range(_K):
                    k = g * _K + s
                    take = v > ts[k]
                    nt = jnp.where(take, v, ts[k])
                    nti = jnp.where(take, vi, tis[k])
                    v = jnp.where(take, ts[k], v)
                    vi = jnp.where(take, tis[k], vi)
                    ts[k] = nt
                    tis[k] = nti
            return tuple(ts) + tuple(tis)

        st = lax.fori_loop(0, _N, col_body, tuple(init))
        return st[:2 * _K], st[2 * _K:]

    def topk1(bufa):
        init = ([jnp.full((16,), _NEG, jnp.float32) for _ in range(_K)]
                + [jnp.zeros((16,), jnp.int32) for _ in range(_K)])

        def col_body(j, st):
            ts = list(st[:_K])
            tis = list(st[_K:])
            jv = jnp.full((16,), j, jnp.int32)
            v = bufa[j]
            vi = jv
            for s in range(_K):
                take = v > ts[s]
                nt = jnp.where(take, v, ts[s])
                nti = jnp.where(take, vi, tis[s])
                v = jnp.where(take, ts[s], v)
                vi = jnp.where(take, tis[s], vi)
                ts[s] = nt
                tis[s] = nti
            return tuple(ts) + tuple(tis)

        st = lax.fori_loop(0, _N, col_body, tuple(init))
        return st[:_K], st[_K:]

    def edge_weights(grp, ts, tis):
        g0 = grp * 16
        pstart = (grp >> 4) * _N
        n_vec = (g0 - pstart) + iot
        rn = n_vec >> 4
        cn = n_vec & 15
        wes = []
        for s in range(_K):
            ri = tis[s] >> 4
            ci = tis[s] & 15
            dr = rn - ri
            dc = cn - ci
            d2 = (dr * dr + dc * dc).astype(jnp.float32)
            sd = jnp.exp(d2 * P(0))
            sf = (ts[s] - sd * P(1)) * P(2)
            tot = P(19)
            for i in range(4):
                h = sf * P(3 + 2 * i) + sd * P(4 + 2 * i) + P(11 + i)
                h = h / (1.0 + jnp.exp(-h))
                tot = tot + h * P(15 + i)
            wes.append(1.0 / (1.0 + jnp.exp(-tot)))
        wsum = wes[0]
        for s in range(1, _K):
            wsum = wsum + wes[s]
        winv = 1.0 / (wsum + 1e-12)
        pbase_vec = jnp.full((16,), pstart, jnp.int32)
        return [we * winv for we in wes], pbase_vec

    def issue_gather(pbase_vec, tis, idxa, idxb, rowsg):
        for s in range(5):
            idxa[pl.ds(s * 16, 16)] = pbase_vec + tis[s]
        for s in range(5, _K):
            idxb[pl.ds((s - 5) * 16, 16)] = pbase_vec + tis[s]
        cp_a = pltpu.async_copy(xflat_hbm.at[idxa], rowsg.at[pl.ds(0, 80)], semg)
        cp_b = pltpu.async_copy(xflat_hbm.at[idxb], rowsg.at[pl.ds(80, 64)], semg)
        return cp_a, cp_b

    def aggregate(grp, wns, rowsg, outv, semo):
        for l in range(16):
            w0 = jnp.full((16,), wns[0][l], jnp.float32)
            a0 = w0 * rowsg[l, pl.ds(0, 16)]
            a1 = w0 * rowsg[l, pl.ds(16, 16)]
            for s in range(1, _K):
                wv = jnp.full((16,), wns[s][l], jnp.float32)
                a0 = a0 + wv * rowsg[s * 16 + l, pl.ds(0, 16)]
                a1 = a1 + wv * rowsg[s * 16 + l, pl.ds(16, 16)]
            outv[l, pl.ds(0, 16)] = a0
            outv[l, pl.ds(16, 16)] = a1
        pltpu.async_copy(outv, out_hbm.at[pl.ds(grp * 16, 16)], semo)

    def drain_out(outv, semo):
        pltpu.make_async_copy(outv, out_hbm.at[pl.ds(0, 16)], semo).wait()

    def process_pair(i, bufa, bufb, grpa, grpb, guard_first):
        ts2, tis2 = topk2(bufa, bufb)
        wns_a, pb_a = edge_weights(grpa, ts2[:_K], tis2[:_K])
        wns_b, pb_b = edge_weights(grpb, ts2[_K:], tis2[_K:])
        cpa0, cpa1 = issue_gather(pb_a, tis2[:_K], idx0a, idx0b, rowsg0)
        cpb0, cpb1 = issue_gather(pb_b, tis2[_K:], idx1a, idx1b, rowsg1)
        cpa0.wait()
        cpa1.wait()

        if guard_first:
            @pl.when(i > 0)
            def _():
                drain_out(outvA, semoA)
        else:
            drain_out(outvA, semoA)

        aggregate(grpa, wns_a, rowsg0, outvA, semoA)
        cpb0.wait()
        cpb1.wait()

        if guard_first:
            @pl.when(i > 0)
            def _():
                drain_out(outvB, semoB)
        else:
            drain_out(outvB, semoB)

        aggregate(grpb, wns_b, rowsg1, outvB, semoB)

    cpA0 = pltpu.async_copy(grp_src(wid), bta0, semA)
    cpA1 = pltpu.async_copy(grp_src(wid + 32), btb0, semA)
    cpB0 = pltpu.async_copy(grp_src(wid + 64), bta1, semB)
    cpB1 = pltpu.async_copy(grp_src(wid + 96), btb1, semB)
    cpA0.wait()
    cpA1.wait()

    def iter_body(i, carry):
        ga = wid + 128 * i
        process_pair(i, bta0, btb0, ga, ga + 32, True)

        @pl.when(i < 5)
        def _():
            pltpu.async_copy(grp_src(ga + 128), bta0, semA)
            pltpu.async_copy(grp_src(ga + 160), btb0, semA)

        pltpu.make_async_copy(grp_src(wid), bta1, semB).wait()
        pltpu.make_async_copy(grp_src(wid), btb1, semB).wait()
        process_pair(i, bta1, btb1, ga + 64, ga + 96, False)

        @pl.when(i < 5)
        def _():
            pltpu.async_copy(grp_src(ga + 192), bta1, semB)
            pltpu.async_copy(grp_src(ga + 224), btb1, semB)

        @pl.when(i < 5)
        def _():
            pltpu.make_async_copy(grp_src(wid), bta0, semA).wait()
            pltpu.make_async_copy(grp_src(wid), btb0, semA).wait()
        return carry

    lax.fori_loop(0, 6, iter_body, 0)

    @pl.when(wid < 16)
    def _():
        pltpu.sync_copy(grp_src(wid + 768), bta0)
        ts1, tis1 = topk1(bta0)
        wns_l, pb_l = edge_weights(wid + 768, ts1, tis1)
        cpl0, cpl1 = issue_gather(pb_l, tis1, idx0a, idx0b, rowsg0)
        cpl0.wait()
        cpl1.wait()
        drain_out(outvA, semoA)
        aggregate(wid + 768, wns_l, rowsg0, outvA, semoA)

    drain_out(outvA, semoA)
    drain_out(outvB, semoB)


_sc_mid = functools.partial(
    pl.kernel,
    out_type=jax.ShapeDtypeStruct((_ROWS, _D4), jnp.float32),
    mesh=plsc.VectorSubcoreMesh(core_axis_name="c", subcore_axis_name="s"),
    compiler_params=pltpu.CompilerParams(use_tc_tiling_on_sc=False,
                                         needs_layout_passes=False),
    scratch_types=[
        pltpu.VMEM((24, 16), jnp.float32),
        pltpu.VMEM((_N, 16), jnp.float32),
        pltpu.VMEM((_N, 16), jnp.float32),
        pltpu.VMEM((_N, 16), jnp.float32),
        pltpu.VMEM((_N, 16), jnp.float32),
        pltpu.VMEM((80,), jnp.int32),
        pltpu.VMEM((64,), jnp.int32),
        pltpu.VMEM((80,), jnp.int32),
        pltpu.VMEM((64,), jnp.int32),
        pltpu.VMEM((144, _D4), jnp.float32),
        pltpu.VMEM((144, _D4), jnp.float32),
        pltpu.VMEM((16, _D4), jnp.float32),
        pltpu.VMEM((16, _D4), jnp.float32),
        pltpu.SemaphoreType.DMA,
        pltpu.SemaphoreType.DMA,
        pltpu.SemaphoreType.DMA,
        pltpu.SemaphoreType.DMA,
        pltpu.SemaphoreType.DMA,
    ],
)(_sc_mid_body)


@jax.jit
def kernel(x_in, sigma, alpha, f_w, f_b, p_w, p_b, mlp_w1, mlp_b1, mlp_w2, mlp_b2):
    B, C, H, Wd = x_in.shape
    ws = 7
    scal = jnp.stack([sigma, alpha]).reshape(1, 2).astype(jnp.float32)
    pvec = jnp.concatenate([
        jnp.stack([
            -1.0 / (2.0 * sigma * sigma),
            1.0 - alpha,
            1.0 / alpha,
        ]),
        mlp_w1.reshape(-1), mlp_b1.reshape(-1),
        mlp_w2.reshape(-1), mlp_b2.reshape(-1),
        jnp.zeros((4,), jnp.float32),
    ]).astype(jnp.float32)
    params = jnp.tile(pvec.reshape(-1, 1), (1, 16))

    xp = x_in.reshape(C, ws, _HP, ws, _HP).transpose(1, 3, 0, 2, 4).reshape(_NP, C, _N)

    comb, xflat = pl.pallas_call(
        _sim_body,
        grid=(_NP,),
        in_specs=[
            pl.BlockSpec((1, 2), lambda p: (0, 0), memory_space=pltpu.SMEM),
            pl.BlockSpec((1, C, _N), lambda p: (p, 0, 0)),
            pl.BlockSpec((_D4, C), lambda p: (0, 0)),
            pl.BlockSpec((_D4, 1), lambda p: (0, 0)),
        ],
        out_specs=[
            pl.BlockSpec((1, _N, _N), lambda p: (p, 0, 0)),
            pl.BlockSpec((1, _N, _D4), lambda p: (p, 0, 0)),
        ],
        out_shape=[
            jax.ShapeDtypeStruct((_NP, _N, _N), jnp.float32),
            jax.ShapeDtypeStruct((_NP, _N, _D4), jnp.float32),
        ],
        scratch_shapes=[pltpu.VMEM((_N, _N), jnp.float32)],
    )(scal, xp, f_w, f_b.reshape(_D4, 1))

    out32 = _sc_mid(comb.reshape(_ROWS, _N), xflat.reshape(_ROWS, _D4), params)

    out = pl.pallas_call(
        _proj_body,
        grid=(_NP,),
        in_specs=[
            pl.BlockSpec((1, _N, _D4), lambda p: (p, 0, 0)),
            pl.BlockSpec((C, _D4), lambda p: (0, 0)),
            pl.BlockSpec((C, 1), lambda p: (0, 0)),
        ],
        out_specs=pl.BlockSpec((1, C, _N), lambda p: (p, 0, 0)),
        out_shape=jax.ShapeDtypeStruct((_NP, C, _N), jnp.float32),
    )(out32.reshape(_NP, _N, _D4), p_w, p_b.reshape(C, 1))

    out = out.reshape(ws, ws, C, _HP, _HP).transpose(2, 0, 3, 1, 4).reshape(B, C, H * Wd)
    return out


# speedup vs baseline: 1.7417x; 1.1398x over previous
import functools
import jax
import jax.numpy as jnp
from jax import lax
from jax.experimental import pallas as pl
from jax.experimental.pallas import tpu as pltpu
from jax.experimental.pallas import tpu_sc as plsc

_HP = 16
_N = _HP * _HP
_NP = 49
_ROWS = _NP * _N
_K = 9
_D4 = 32
_NEG = -3.0e38

_NW = 32
_RPW = _ROWS // _NW
_NG = 25


def _sim_body(scal_ref, x_ref, fw_ref, fb_ref, comb_ref, xflat_ref, d_scr):
    p = pl.program_id(0)
    sigma = scal_ref[0, 0]
    alpha = scal_ref[0, 1]

    @pl.when(p == 0)
    def _():
        ni = lax.broadcasted_iota(jnp.int32, (_N, _N), 0)
        mi = lax.broadcasted_iota(jnp.int32, (_N, _N), 1)
        dr = (ni // _HP) - (mi // _HP)
        dc = (ni % _HP) - (mi % _HP)
        d2 = (dr * dr + dc * dc).astype(jnp.float32)
        d_scr[...] = jnp.exp(d2 * (-1.0 / (2.0 * sigma * sigma)))

    Dm = d_scr[...]
    for c in range(7):
        xm = x_ref[0, :, :, c * _HP:(c + 1) * _HP].reshape(128, _N)
        ft = lax.dot_general(fw_ref[...], xm, (((1,), (0,)), ((), ())),
                             preferred_element_type=jnp.float32)
        ft = ft + fb_ref[...]
        nsq = jnp.sum(ft * ft, axis=0)[None, :]
        inv = 1.0 / jnp.maximum(jnp.sqrt(nsq), 1e-8)
        ftn = ft * inv
        S = lax.dot_general(ftn, ftn, (((0,), (0,)), ((), ())),
                            preferred_element_type=jnp.float32)
        comb_ref[c] = alpha * S + (1.0 - alpha) * Dm
        xflat_ref[c] = ft.T


def _proj_body(agg_ref, pw_ref, pb_ref, out_ref):
    for c in range(7):
        y = lax.dot_general(pw_ref[...], agg_ref[c], (((1,), (1,)), ((), ())),
                            preferred_element_type=jnp.float32)
        y = y + pb_ref[...]
        out_ref[0, :, :, c * _HP:(c + 1) * _HP] = y.reshape(128, _HP, _HP)


def _sc_mid_body(comb_hbm, xflat_hbm, params_hbm, out_hbm,
                 params_v, bta0, btb0, bta1, btb1,
                 idx0a, idx0b, idx1a, idx1b,
                 rowsg0, rowsg1, outvA, outvB, semA, semB, semg, semoA, semoB):
    wid = lax.axis_index("s") * 2 + lax.axis_index("c")
    pltpu.sync_copy(params_hbm, params_v)

    def P(i):
        return params_v[i]

    iot = lax.iota(jnp.int32, 16)

    def grp_src(grp):
        g0 = grp * 16
        pstart = (grp >> 4) * _N
        n0 = g0 - pstart
        return comb_hbm.at[pl.ds(pstart, _N), pl.ds(n0, 16)]

    def topk2(bufa, bufb):
        init = ([jnp.full((16,), _NEG, jnp.float32) for _ in range(2 * _K)]
                + [jnp.zeros((16,), jnp.int32) for _ in range(2 * _K)])

        def col_body(j, st):
            ts = list(st[:2 * _K])
            tis = list(st[2 * _K:])
            jv = jnp.full((16,), j, jnp.int32)
            vs = [bufa[j], bufb[j]]
            vis = [jv, jv]
            for g in range(2):
                v = vs[g]
                vi = vis[g]
                for s in range(_K):
                    k = g * _K + s
                    take = v > ts[k]
                    nt = jnp.where(take, v, ts[k])
                    nti = jnp.where(take, vi, tis[k])
                    v = jnp.where(take, ts[k], v)
                    vi = jnp.where(take, tis[k], vi)
                    ts[k] = nt
                    tis[k] = nti
            return tuple(ts) + tuple(tis)

        st = lax.fori_loop(0, _N, col_body, tuple(init))
        return st[:2 * _K], st[2 * _K:]

    def topk1(bufa):
        init = ([jnp.full((16,), _NEG, jnp.float32) for _ in range(_K)]
                + [jnp.zeros((16,), jnp.int32) for _ in range(_K)])

        def col_body(j, st):
            ts = list(st[:_K])
            tis = list(st[_K:])
            jv = jnp.full((16,), j, jnp.int32)
            v = bufa[j]
            vi = jv
            for s in range(_K):
                take = v > ts[s]
                nt = jnp.where(take, v, ts[s])
                nti = jnp.where(take, vi, tis[s])
                v = jnp.where(take, ts[s], v)
                vi = jnp.where(take, tis[s], vi)
                ts[s] = nt
                tis[s] = nti
            return tuple(ts) + tuple(tis)

        st = lax.fori_loop(0, _N, col_body, tuple(init))
        return st[:_K], st[_K:]

    def edge_weights(grp, ts, tis):
        g0 = grp * 16
        pstart = (grp >> 4) * _N
        n_vec = (g0 - pstart) + iot
        rn = n_vec >> 4
        cn = n_vec & 15
        wes = []
        for s in range(_K):
            ri = tis[s] >> 4
            ci = tis[s] & 15
            dr = rn - ri
            dc = cn - ci
            d2 = (dr * dr + dc * dc).astype(jnp.float32)
            sd = jnp.exp(d2 * P(0))
            sf = (ts[s] - sd * P(1)) * P(2)
            tot = P(19)
            for i in range(4):
                h = sf * P(3 + 2 * i) + sd * P(4 + 2 * i) + P(11 + i)
                h = h / (1.0 + jnp.exp(-h))
                tot = tot + h * P(15 + i)
            wes.append(1.0 / (1.0 + jnp.exp(-tot)))
        wsum = wes[0]
        for s in range(1, _K):
            wsum = wsum + wes[s]
        winv = 1.0 / (wsum + 1e-12)
        pbase_vec = jnp.full((16,), pstart, jnp.int32)
        return [we * winv for we in wes], pbase_vec

    def issue_gather(pbase_vec, tis, idxa, idxb, rowsg):
        for s in range(5):
            idxa[pl.ds(s * 16, 16)] = pbase_vec + tis[s]
        for s in range(5, _K):
            idxb[pl.ds((s - 5) * 16, 16)] = pbase_vec + tis[s]
        cp_a = pltpu.async_copy(xflat_hbm.at[idxa], rowsg.at[pl.ds(0, 80)], semg)
        cp_b = pltpu.async_copy(xflat_hbm.at[idxb], rowsg.at[pl.ds(80, 64)], semg)
        return cp_a, cp_b

    def aggregate(grp, wns, rowsg, outv, semo):
        for l in range(16):
            w0 = jnp.full((16,), wns[0][l], jnp.float32)
            a0 = w0 * rowsg[l, pl.ds(0, 16)]
            a1 = w0 * rowsg[l, pl.ds(16, 16)]
            for s in range(1, _K):
                wv = jnp.full((16,), wns[s][l], jnp.float32)
                a0 = a0 + wv * rowsg[s * 16 + l, pl.ds(0, 16)]
                a1 = a1 + wv * rowsg[s * 16 + l, pl.ds(16, 16)]
            outv[l, pl.ds(0, 16)] = a0
            outv[l, pl.ds(16, 16)] = a1
        pltpu.async_copy(outv, out_hbm.at[pl.ds(grp * 16, 16)], semo)

    def drain_out(outv, semo):
        pltpu.make_async_copy(outv, out_hbm.at[pl.ds(0, 16)], semo).wait()

    def process_pair(i, bufa, bufb, grpa, grpb, guard_first):
        ts2, tis2 = topk2(bufa, bufb)
        wns_a, pb_a = edge_weights(grpa, ts2[:_K], tis2[:_K])
        wns_b, pb_b = edge_weights(grpb, ts2[_K:], tis2[_K:])
        cpa0, cpa1 = issue_gather(pb_a, tis2[:_K], idx0a, idx0b, rowsg0)
        cpb0, cpb1 = issue_gather(pb_b, tis2[_K:], idx1a, idx1b, rowsg1)
        cpa0.wait()
        cpa1.wait()

        if guard_first:
            @pl.when(i > 0)
            def _():
                drain_out(outvA, semoA)
        else:
            drain_out(outvA, semoA)

        aggregate(grpa, wns_a, rowsg0, outvA, semoA)
        cpb0.wait()
        cpb1.wait()

        if guard_first:
            @pl.when(i > 0)
            def _():
                drain_out(outvB, semoB)
        else:
            drain_out(outvB, semoB)

        aggregate(grpb, wns_b, rowsg1, outvB, semoB)

    cpA0 = pltpu.async_copy(grp_src(wid), bta0, semA)
    cpA1 = pltpu.async_copy(grp_src(wid + 32), btb0, semA)
    cpB0 = pltpu.async_copy(grp_src(wid + 64), bta1, semB)
    cpB1 = pltpu.async_copy(grp_src(wid + 96), btb1, semB)
    cpA0.wait()
    cpA1.wait()

    def iter_body(i, carry):
        ga = wid + 128 * i
        process_pair(i, bta0, btb0, ga, ga + 32, True)

        @pl.when(i < 5)
        def _():
            pltpu.async_copy(grp_src(ga + 128), bta0, semA)
            pltpu.async_copy(grp_src(ga + 160), btb0, semA)

        pltpu.make_async_copy(grp_src(wid), bta1, semB).wait()
        pltpu.make_async_copy(grp_src(wid), btb1, semB).wait()
        process_pair(i, bta1, btb1, ga + 64, ga + 96, False)

        @pl.when(i < 5)
        def _():
            pltpu.async_copy(grp_src(ga + 192), bta1, semB)
            pltpu.async_copy(grp_src(ga + 224), btb1, semB)

        @pl.when(i < 5)
        def _():
            pltpu.make_async_copy(grp_src(wid), bta0, semA).wait()
            pltpu.make_async_copy(grp_src(wid), btb0, semA).wait()
        return carry

    lax.fori_loop(0, 6, iter_body, 0)

    @pl.when(wid < 16)
    def _():
        pltpu.sync_copy(grp_src(wid + 768), bta0)
        ts1, tis1 = topk1(bta0)
        wns_l, pb_l = edge_weights(wid + 768, ts1, tis1)
        cpl0, cpl1 = issue_gather(pb_l, tis1, idx0a, idx0b, rowsg0)
        cpl0.wait()
        cpl1.wait()
        drain_out(outvA, semoA)
        aggregate(wid + 768, wns_l, rowsg0, outvA, semoA)

    drain_out(outvA, semoA)
    drain_out(outvB, semoB)


_sc_mid = functools.partial(
    pl.kernel,
    out_type=jax.ShapeDtypeStruct((_ROWS, _D4), jnp.float32),
    mesh=plsc.VectorSubcoreMesh(core_axis_name="c", subcore_axis_name="s"),
    compiler_params=pltpu.CompilerParams(use_tc_tiling_on_sc=False,
                                         needs_layout_passes=False),
    scratch_types=[
        pltpu.VMEM((24, 16), jnp.float32),
        pltpu.VMEM((_N, 16), jnp.float32),
        pltpu.VMEM((_N, 16), jnp.float32),
        pltpu.VMEM((_N, 16), jnp.float32),
        pltpu.VMEM((_N, 16), jnp.float32),
        pltpu.VMEM((80,), jnp.int32),
        pltpu.VMEM((64,), jnp.int32),
        pltpu.VMEM((80,), jnp.int32),
        pltpu.VMEM((64,), jnp.int32),
        pltpu.VMEM((144, _D4), jnp.float32),
        pltpu.VMEM((144, _D4), jnp.float32),
        pltpu.VMEM((16, _D4), jnp.float32),
        pltpu.VMEM((16, _D4), jnp.float32),
        pltpu.SemaphoreType.DMA,
        pltpu.SemaphoreType.DMA,
        pltpu.SemaphoreType.DMA,
        pltpu.SemaphoreType.DMA,
        pltpu.SemaphoreType.DMA,
    ],
)(_sc_mid_body)


@jax.jit
def kernel(x_in, sigma, alpha, f_w, f_b, p_w, p_b, mlp_w1, mlp_b1, mlp_w2, mlp_b2):
    B, C, H, Wd = x_in.shape
    ws = 7
    scal = jnp.stack([sigma, alpha]).reshape(1, 2).astype(jnp.float32)
    pvec = jnp.concatenate([
        jnp.stack([
            -1.0 / (2.0 * sigma * sigma),
            1.0 - alpha,
            1.0 / alpha,
        ]),
        mlp_w1.reshape(-1), mlp_b1.reshape(-1),
        mlp_w2.reshape(-1), mlp_b2.reshape(-1),
        jnp.zeros((4,), jnp.float32),
    ]).astype(jnp.float32)
    params = jnp.tile(pvec.reshape(-1, 1), (1, 16))

    comb, xflat = pl.pallas_call(
        _sim_body,
        grid=(ws,),
        in_specs=[
            pl.BlockSpec((1, 2), lambda a: (0, 0), memory_space=pltpu.SMEM),
            pl.BlockSpec((1, C, _HP, H), lambda a: (0, 0, a, 0)),
            pl.BlockSpec((_D4, C), lambda a: (0, 0)),
            pl.BlockSpec((_D4, 1), lambda a: (0, 0)),
        ],
        out_specs=[
            pl.BlockSpec((ws, _N, _N), lambda a: (a, 0, 0)),
            pl.BlockSpec((ws, _N, _D4), lambda a: (a, 0, 0)),
        ],
        out_shape=[
            jax.ShapeDtypeStruct((_NP, _N, _N), jnp.float32),
            jax.ShapeDtypeStruct((_NP, _N, _D4), jnp.float32),
        ],
        scratch_shapes=[pltpu.VMEM((_N, _N), jnp.float32)],
    )(scal, x_in, f_w, f_b.reshape(_D4, 1))

    out32 = _sc_mid(comb.reshape(_ROWS, _N), xflat.reshape(_ROWS, _D4), params)

    out = pl.pallas_call(
        _proj_body,
        grid=(ws,),
        in_specs=[
            pl.BlockSpec((ws, _N, _D4), lambda a: (a, 0, 0)),
            pl.BlockSpec((C, _D4), lambda a: (0, 0)),
            pl.BlockSpec((C, 1), lambda a: (0, 0)),
        ],
        out_specs=pl.BlockSpec((1, C, _HP, H), lambda a: (0, 0, a, 0)),
        out_shape=jax.ShapeDtypeStruct((B, C, H, Wd), jnp.float32),
    )(out32.reshape(_NP, _N, _D4), p_w, p_b.reshape(C, 1))

    return out.reshape(B, C, H * Wd)
